# baseline (speedup 1.0000x reference)
import functools

import jax
import jax.numpy as jnp
from jax import lax
from jax.experimental import pallas as pl
from jax.experimental.pallas import tpu as pltpu
from jax.experimental.pallas import tpu_sc as plsc

N = 10000
E = 160000
PAD_N = 10016
NC, NS = 2, 16
NW = NC * NS
E_PAD = 163840
EPT = E_PAD // NW
CH = 128
NCH = EPT // CH
ROWS_I = 632
ROWS_I_LAST = N - 15 * ROWS_I
ROWS_D = 632
ROWS_D_LAST = PAD_N - 15 * ROWS_D


def _sc_mesh():
    return plsc.VectorSubcoreMesh(core_axis_name="c", subcore_axis_name="s",
                                  num_cores=NC, num_subcores=NS)


def _acc_init(s, zero_hbm, acc_sh):
    @pl.when(s < NS - 1)
    def _():
        pltpu.sync_copy(zero_hbm.at[pl.ds(s * ROWS_I, ROWS_I)],
                        acc_sh.at[pl.ds(s * ROWS_I, ROWS_I)])

    @pl.when(s == NS - 1)
    def _():
        pltpu.sync_copy(zero_hbm.at[pl.ds(15 * ROWS_I, ROWS_I_LAST)],
                        acc_sh.at[pl.ds(15 * ROWS_I, ROWS_I_LAST)])


def _acc_dump(c, s, acc_sh, out_hbm):
    @pl.when(s < NS - 1)
    def _():
        pltpu.sync_copy(acc_sh.at[pl.ds(s * ROWS_D, ROWS_D)],
                        out_hbm.at[c].at[pl.ds(s * ROWS_D, ROWS_D)])

    @pl.when(s == NS - 1)
    def _():
        pltpu.sync_copy(acc_sh.at[pl.ds(15 * ROWS_D, ROWS_D_LAST)],
                        out_hbm.at[c].at[pl.ds(15 * ROWS_D, ROWS_D_LAST)])


def _make_agg(F, NB):
    @functools.partial(
        pl.kernel,
        out_type=jax.ShapeDtypeStruct((NC, PAD_N, F), jnp.float32),
        mesh=_sc_mesh(),
        scratch_types=(
            [pltpu.VMEM((NCH, CH), jnp.int32),
             pltpu.VMEM((NCH, CH), jnp.int32)]
            + [pltpu.VMEM((CH, F), jnp.float32)] * NB
            + [pltpu.VMEM_SHARED((PAD_N, F), jnp.float32)]
            + [pltpu.SemaphoreType.DMA] * NB
        ),
        compiler_params=pltpu.CompilerParams(use_tc_tiling_on_sc=False),
    )
    def agg(y_hbm, src_hbm, dst_hbm, zero_hbm, out_hbm,
            sidx_v, didx_v, *rest):
        rows = rest[:NB]
        acc_sh = rest[NB]
        sems = rest[NB + 1:]
        c = lax.axis_index("c")
        s = lax.axis_index("s")
        w = c * NS + s
        pltpu.sync_copy(src_hbm.at[pl.ds(w * NCH, NCH)], sidx_v)
        pltpu.sync_copy(dst_hbm.at[pl.ds(w * NCH, NCH)], didx_v)
        _acc_init(s, zero_hbm, acc_sh)
        plsc.subcore_barrier()

        for b in range(NB - 1):
            pltpu.async_copy(y_hbm.at[sidx_v.at[b]], rows[b], sems[b])

        def body(g, carry):
            for b in range(NB):
                i = g * NB + b
                pltpu.make_async_copy(y_hbm.at[sidx_v.at[i]],
                                      rows[b], sems[b]).wait()

                @pl.when(i + NB - 1 < NCH)
                def _():
                    nb = (b + NB - 1) % NB
                    pltpu.async_copy(y_hbm.at[sidx_v.at[i + NB - 1]],
                                     rows[nb], sems[nb])

                pltpu.sync_copy(rows[b], acc_sh.at[didx_v.at[i]], add=True)
            return carry

        lax.fori_loop(0, NCH // NB, body, 0)
        plsc.subcore_barrier()
        _acc_dump(c, s, acc_sh, out_hbm)

    return agg


def _make_deg():
    @functools.partial(
        pl.kernel,
        out_type=jax.ShapeDtypeStruct((NC, PAD_N, 16), jnp.float32),
        mesh=_sc_mesh(),
        scratch_types=[
            pltpu.VMEM((NCH, CH), jnp.int32),
            pltpu.VMEM((CH, 16), jnp.float32),
            pltpu.VMEM_SHARED((PAD_N, 16), jnp.float32),
        ],
        compiler_params=pltpu.CompilerParams(use_tc_tiling_on_sc=False),
    )
    def deg(dst_hbm, ones_hbm, zero_hbm, out_hbm, didx_v, ones_v, acc_sh):
        c = lax.axis_index("c")
        s = lax.axis_index("s")
        w = c * NS + s
        pltpu.sync_copy(ones_hbm, ones_v)
        pltpu.sync_copy(dst_hbm.at[pl.ds(w * NCH, NCH)], didx_v)
        _acc_init(s, zero_hbm, acc_sh)
        plsc.subcore_barrier()

        def body(i, carry):
            pltpu.sync_copy(ones_v, acc_sh.at[didx_v.at[i]], add=True)
            return carry

        lax.fori_loop(0, NCH, body, 0)
        plsc.subcore_barrier()
        _acc_dump(c, s, acc_sh, out_hbm)

    return deg


def _tc_stage_a(latent, hs, Wl, Wt, degp):
    R = 400
    G = N // R

    def body(lat_ref, hs_ref, wl_ref, wt_ref, deg_ref, y_ref, dinv_ref):
        t = jnp.dot(lat_ref[...], wl_ref[...], preferred_element_type=jnp.float32)
        t = t + jnp.dot(hs_ref[...], wt_ref[...], preferred_element_type=jnp.float32)
        dp = deg_ref[...]
        deg = 1.0 + dp[0, :, 0:1] + dp[1, :, 0:1]
        dinv = lax.rsqrt(deg)
        y_ref[...] = t * dinv
        dinv_ref[...] = dinv

    return pl.pallas_call(
        body,
        grid=(G,),
        in_specs=[
            pl.BlockSpec((R, 256), lambda i: (i, 0)),
            pl.BlockSpec((R, 2), lambda i: (i, 0)),
            pl.BlockSpec((256, 128), lambda i: (0, 0)),
            pl.BlockSpec((2, 128), lambda i: (0, 0)),
            pl.BlockSpec((NC, R, 16), lambda i: (0, i, 0)),
        ],
        out_specs=[
            pl.BlockSpec((R, 128), lambda i: (i, 0)),
            pl.BlockSpec((R, 1), lambda i: (i, 0)),
        ],
        out_shape=[
            jax.ShapeDtypeStruct((N, 128), jnp.float32),
            jax.ShapeDtypeStruct((N, 1), jnp.float32),
        ],
    )(latent, hs, Wl, Wt, degp)


def _tc_stage_mid(acc, y, dinv, b, Wn, F, F2, F2_store):
    R = 400
    G = N // R

    def body(acc_ref, y_ref, dinv_ref, b_ref, w_ref, out_ref):
        a = acc_ref[...]
        dinv = dinv_ref[...]
        h = (a[0] + a[1] + y_ref[...]) * dinv + b_ref[...]
        h = jnp.maximum(h, 0.0)
        yn = jnp.dot(h, w_ref[...], preferred_element_type=jnp.float32) * dinv
        if F2_store != F2:
            yn = jnp.pad(yn, ((0, 0), (0, F2_store - F2)))
        out_ref[...] = yn

    return pl.pallas_call(
        body,
        grid=(G,),
        in_specs=[
            pl.BlockSpec((NC, R, F), lambda i: (0, i, 0)),
            pl.BlockSpec((R, F), lambda i: (i, 0)),
            pl.BlockSpec((R, 1), lambda i: (i, 0)),
            pl.BlockSpec((1, F), lambda i: (0, 0)),
            pl.BlockSpec((F, F2), lambda i: (0, 0)),
        ],
        out_specs=pl.BlockSpec((R, F2_store), lambda i: (i, 0)),
        out_shape=jax.ShapeDtypeStruct((N, F2_store), jnp.float32),
    )(acc, y, dinv, b, Wn)


def _tc_stage_final(acc, y, dinv, b3, Wc, bc):
    R = 400
    G = N // R

    def body(acc_ref, y_ref, dinv_ref, b_ref, w_ref, bc_ref, out_ref):
        a = acc_ref[...]
        dinv = dinv_ref[...]
        h = (a[0, :, 0:8] + a[1, :, 0:8] + y_ref[..., 0:8]) * dinv + b_ref[...]
        h = jnp.maximum(h, 0.0)
        out_ref[...] = (
            jnp.dot(h, w_ref[...], preferred_element_type=jnp.float32) + bc_ref[...]
        )

    return pl.pallas_call(
        body,
        grid=(G,),
        in_specs=[
            pl.BlockSpec((NC, R, 16), lambda i: (0, i, 0)),
            pl.BlockSpec((R, 16), lambda i: (i, 0)),
            pl.BlockSpec((R, 1), lambda i: (i, 0)),
            pl.BlockSpec((1, 8), lambda i: (0, 0)),
            pl.BlockSpec((8, 3), lambda i: (0, 0)),
            pl.BlockSpec((1, 3), lambda i: (0, 0)),
        ],
        out_specs=pl.BlockSpec((R, 3), lambda i: (i, 0)),
        out_shape=jax.ShapeDtypeStruct((N, 3), jnp.float32),
    )(acc, y, dinv, b3, Wc, bc)


def kernel(latent_features, history, is_subgoal, edge_data,
           W1, b1, W2, b2, W3, b3, Wc, bc):
    src = edge_data[0].astype(jnp.int32)
    dst = edge_data[1].astype(jnp.int32)
    src_p = jnp.concatenate(
        [src, jnp.zeros((E_PAD - E,), jnp.int32)]).reshape(E_PAD // CH, CH)
    dst_p = jnp.concatenate(
        [dst, jnp.full((E_PAD - E,), N, jnp.int32)]).reshape(E_PAD // CH, CH)

    zeros128 = jnp.zeros((N, 128), jnp.float32)
    zeros64 = jnp.zeros((N, 64), jnp.float32)
    zeros16 = jnp.zeros((N, 16), jnp.float32)
    ones16 = jnp.ones((CH, 16), jnp.float32)

    degp = _make_deg()(dst_p, ones16, zeros16)

    hs = jnp.stack([history, is_subgoal], axis=1)
    y1, dinv = _tc_stage_a(latent_features, hs, W1[:256], W1[256:258], degp)

    acc1 = _make_agg(128, 2)(y1, src_p, dst_p, zeros128)
    y2 = _tc_stage_mid(acc1, y1, dinv, b1.reshape(1, -1), W2, 128, 64, 64)
    acc2 = _make_agg(64, 4)(y2, src_p, dst_p, zeros64)
    y3 = _tc_stage_mid(acc2, y2, dinv, b2.reshape(1, -1), W3, 64, 8, 16)
    acc3 = _make_agg(16, 4)(y3, src_p, dst_p, zeros16)
    out = _tc_stage_final(acc3, y3, dinv, b3.reshape(1, -1), Wc, bc.reshape(1, -1))
    return out


# speedup vs baseline: 1.1269x; 1.1269x over previous
import functools

import jax
import jax.numpy as jnp
from jax import lax
from jax.experimental import pallas as pl
from jax.experimental.pallas import tpu as pltpu
from jax.experimental.pallas import tpu_sc as plsc

N = 10000
E = 160000
PAD_N = 10016
NC, NS = 2, 16
NW = NC * NS
E_PAD = 163840
EPT = E_PAD // NW
CH = 128
NCH = EPT // CH
ROWS_I = 632
ROWS_I_LAST = N - 15 * ROWS_I
ROWS_D = 632
ROWS_D_LAST = PAD_N - 15 * ROWS_D


def _sc_mesh():
    return plsc.VectorSubcoreMesh(core_axis_name="c", subcore_axis_name="s",
                                  num_cores=NC, num_subcores=NS)


def _acc_init(s, zero_hbm, acc_sh):
    @pl.when(s < NS - 1)
    def _():
        pltpu.sync_copy(zero_hbm.at[pl.ds(s * ROWS_I, ROWS_I)],
                        acc_sh.at[pl.ds(s * ROWS_I, ROWS_I)])

    @pl.when(s == NS - 1)
    def _():
        pltpu.sync_copy(zero_hbm.at[pl.ds(15 * ROWS_I, ROWS_I_LAST)],
                        acc_sh.at[pl.ds(15 * ROWS_I, ROWS_I_LAST)])


def _acc_dump(c, s, acc_sh, out_hbm):
    @pl.when(s < NS - 1)
    def _():
        pltpu.sync_copy(acc_sh.at[pl.ds(s * ROWS_D, ROWS_D)],
                        out_hbm.at[c].at[pl.ds(s * ROWS_D, ROWS_D)])

    @pl.when(s == NS - 1)
    def _():
        pltpu.sync_copy(acc_sh.at[pl.ds(15 * ROWS_D, ROWS_D_LAST)],
                        out_hbm.at[c].at[pl.ds(15 * ROWS_D, ROWS_D_LAST)])


def _make_agg(F, NB):
    @functools.partial(
        pl.kernel,
        out_type=jax.ShapeDtypeStruct((NC, PAD_N, F), jnp.float32),
        mesh=_sc_mesh(),
        scratch_types=(
            [pltpu.VMEM((NCH, CH), jnp.int32),
             pltpu.VMEM((NCH, CH), jnp.int32)]
            + [pltpu.VMEM((CH, F), jnp.float32)] * NB
            + [pltpu.VMEM_SHARED((PAD_N, F), jnp.float32)]
            + [pltpu.SemaphoreType.DMA] * NB
        ),
        compiler_params=pltpu.CompilerParams(use_tc_tiling_on_sc=False),
    )
    def agg(y_hbm, src_hbm, dst_hbm, zero_hbm, out_hbm,
            sidx_v, didx_v, *rest):
        rows = rest[:NB]
        acc_sh = rest[NB]
        sems = rest[NB + 1:]
        c = lax.axis_index("c")
        s = lax.axis_index("s")
        w = c * NS + s
        pltpu.sync_copy(src_hbm.at[pl.ds(w * NCH, NCH)], sidx_v)
        pltpu.sync_copy(dst_hbm.at[pl.ds(w * NCH, NCH)], didx_v)
        _acc_init(s, zero_hbm, acc_sh)
        plsc.subcore_barrier()

        for b in range(NB - 1):
            pltpu.async_copy(y_hbm.at[sidx_v.at[b]], rows[b], sems[b])

        def body(g, carry):
            for b in range(NB):
                i = g * NB + b
                pltpu.make_async_copy(y_hbm.at[sidx_v.at[i]],
                                      rows[b], sems[b]).wait()

                @pl.when(i + NB - 1 < NCH)
                def _():
                    nb = (b + NB - 1) % NB
                    pltpu.async_copy(y_hbm.at[sidx_v.at[i + NB - 1]],
                                     rows[nb], sems[nb])

                pltpu.sync_copy(rows[b], acc_sh.at[didx_v.at[i]], add=True)
            return carry

        lax.fori_loop(0, NCH // NB, body, 0)
        plsc.subcore_barrier()
        _acc_dump(c, s, acc_sh, out_hbm)

    return agg


def _make_deg():
    @functools.partial(
        pl.kernel,
        out_type=jax.ShapeDtypeStruct((NC, PAD_N, 16), jnp.float32),
        mesh=_sc_mesh(),
        scratch_types=[
            pltpu.VMEM((NCH, CH), jnp.int32),
            pltpu.VMEM((CH, 16), jnp.float32),
            pltpu.VMEM_SHARED((PAD_N, 16), jnp.float32),
        ],
        compiler_params=pltpu.CompilerParams(use_tc_tiling_on_sc=False),
    )
    def deg(dst_hbm, ones_hbm, zero_hbm, out_hbm, didx_v, ones_v, acc_sh):
        c = lax.axis_index("c")
        s = lax.axis_index("s")
        w = c * NS + s
        pltpu.sync_copy(ones_hbm, ones_v)
        pltpu.sync_copy(dst_hbm.at[pl.ds(w * NCH, NCH)], didx_v)
        _acc_init(s, zero_hbm, acc_sh)
        plsc.subcore_barrier()

        def body(i, carry):
            pltpu.sync_copy(ones_v, acc_sh.at[didx_v.at[i]], add=True)
            return carry

        lax.fori_loop(0, NCH, body, 0)
        plsc.subcore_barrier()
        _acc_dump(c, s, acc_sh, out_hbm)

    return deg


def _tc_stage_a(latent, hs, Wl, Wt, degp):
    R = 400
    G = N // R

    def body(lat_ref, hs_ref, wl_ref, wt_ref, deg_ref, y_ref, dinv_ref):
        t = jnp.dot(lat_ref[...], wl_ref[...], preferred_element_type=jnp.float32)
        t = t + jnp.dot(hs_ref[...], wt_ref[...], preferred_element_type=jnp.float32)
        dp = deg_ref[...]
        deg = 1.0 + dp[0, :, 0:1] + dp[1, :, 0:1]
        dinv = lax.rsqrt(deg)
        y_ref[...] = t * dinv
        dinv_ref[...] = dinv

    return pl.pallas_call(
        body,
        grid=(G,),
        in_specs=[
            pl.BlockSpec((R, 256), lambda i: (i, 0)),
            pl.BlockSpec((R, 2), lambda i: (i, 0)),
            pl.BlockSpec((256, 128), lambda i: (0, 0)),
            pl.BlockSpec((2, 128), lambda i: (0, 0)),
            pl.BlockSpec((NC, R, 16), lambda i: (0, i, 0)),
        ],
        out_specs=[
            pl.BlockSpec((R, 128), lambda i: (i, 0)),
            pl.BlockSpec((R, 1), lambda i: (i, 0)),
        ],
        out_shape=[
            jax.ShapeDtypeStruct((N, 128), jnp.float32),
            jax.ShapeDtypeStruct((N, 1), jnp.float32),
        ],
    )(latent, hs, Wl, Wt, degp)


def _tc_stage_mid(acc, y, dinv, b, Wn, F, F2, F2_store):
    R = 400
    G = N // R

    def body(acc_ref, y_ref, dinv_ref, b_ref, w_ref, out_ref):
        a = acc_ref[...]
        dinv = dinv_ref[...]
        h = (a[0] + a[1] + y_ref[...]) * dinv + b_ref[...]
        h = jnp.maximum(h, 0.0)
        yn = jnp.dot(h, w_ref[...], preferred_element_type=jnp.float32) * dinv
        if F2_store != F2:
            yn = jnp.pad(yn, ((0, 0), (0, F2_store - F2)))
        out_ref[...] = yn

    return pl.pallas_call(
        body,
        grid=(G,),
        in_specs=[
            pl.BlockSpec((NC, R, F), lambda i: (0, i, 0)),
            pl.BlockSpec((R, F), lambda i: (i, 0)),
            pl.BlockSpec((R, 1), lambda i: (i, 0)),
            pl.BlockSpec((1, F), lambda i: (0, 0)),
            pl.BlockSpec((F, F2), lambda i: (0, 0)),
        ],
        out_specs=pl.BlockSpec((R, F2_store), lambda i: (i, 0)),
        out_shape=jax.ShapeDtypeStruct((N, F2_store), jnp.float32),
    )(acc, y, dinv, b, Wn)


def _tc_stage_final(acc, y, dinv, b3, Wc, bc):
    R = 400
    G = N // R

    def body(acc_ref, y_ref, dinv_ref, b_ref, w_ref, bc_ref, out_ref):
        a = acc_ref[...]
        dinv = dinv_ref[...]
        h = (a[0, :, 0:8] + a[1, :, 0:8] + y_ref[..., 0:8]) * dinv + b_ref[...]
        h = jnp.maximum(h, 0.0)
        out_ref[...] = (
            jnp.dot(h, w_ref[...], preferred_element_type=jnp.float32) + bc_ref[...]
        )

    return pl.pallas_call(
        body,
        grid=(G,),
        in_specs=[
            pl.BlockSpec((NC, R, 16), lambda i: (0, i, 0)),
            pl.BlockSpec((R, 16), lambda i: (i, 0)),
            pl.BlockSpec((R, 1), lambda i: (i, 0)),
            pl.BlockSpec((1, 8), lambda i: (0, 0)),
            pl.BlockSpec((8, 3), lambda i: (0, 0)),
            pl.BlockSpec((1, 3), lambda i: (0, 0)),
        ],
        out_specs=pl.BlockSpec((R, 3), lambda i: (i, 0)),
        out_shape=jax.ShapeDtypeStruct((N, 3), jnp.float32),
    )(acc, y, dinv, b3, Wc, bc)


def kernel(latent_features, history, is_subgoal, edge_data,
           W1, b1, W2, b2, W3, b3, Wc, bc):
    src = edge_data[0].astype(jnp.int32)
    dst = edge_data[1].astype(jnp.int32)
    n_pad_w = (E_PAD - E) // NW
    pad_dst = jnp.broadcast_to(
        N + (jnp.arange(n_pad_w, dtype=jnp.int32) % 16)[None, :], (NW, n_pad_w))
    src_p = jnp.concatenate(
        [src.reshape(NW, E // NW),
         jnp.zeros((NW, n_pad_w), jnp.int32)], axis=1).reshape(E_PAD // CH, CH)
    dst_p = jnp.concatenate(
        [dst.reshape(NW, E // NW), pad_dst], axis=1).reshape(E_PAD // CH, CH)

    zeros128 = jnp.zeros((N, 128), jnp.float32)
    zeros64 = jnp.zeros((N, 64), jnp.float32)
    zeros16 = jnp.zeros((N, 16), jnp.float32)
    ones16 = jnp.ones((CH, 16), jnp.float32)

    degp = _make_deg()(dst_p, ones16, zeros16)

    hs = jnp.stack([history, is_subgoal], axis=1)
    y1, dinv = _tc_stage_a(latent_features, hs, W1[:256], W1[256:258], degp)

    acc1 = _make_agg(128, 2)(y1, src_p, dst_p, zeros128)
    y2 = _tc_stage_mid(acc1, y1, dinv, b1.reshape(1, -1), W2, 128, 64, 64)
    acc2 = _make_agg(64, 4)(y2, src_p, dst_p, zeros64)
    y3 = _tc_stage_mid(acc2, y2, dinv, b2.reshape(1, -1), W3, 64, 8, 16)
    acc3 = _make_agg(16, 4)(y3, src_p, dst_p, zeros16)
    out = _tc_stage_final(acc3, y3, dinv, b3.reshape(1, -1), Wc, bc.reshape(1, -1))
    return out


# speedup vs baseline: 1.8439x; 1.6363x over previous
import functools

import jax
import jax.numpy as jnp
from jax import lax
from jax.experimental import pallas as pl
from jax.experimental.pallas import tpu as pltpu
from jax.experimental.pallas import tpu_sc as plsc

N = 10000
E = 160000
PAD_N = 10016
NC, NS = 2, 16
NW = NC * NS
E_PAD = 163840
EPT = E_PAD // NW
CH = 128
NCH = EPT // CH
ROWS_I = 632
ROWS_I_LAST = N - 15 * ROWS_I
ROWS_D = 632
ROWS_D_LAST = PAD_N - 15 * ROWS_D


def _sc_mesh():
    return plsc.VectorSubcoreMesh(core_axis_name="c", subcore_axis_name="s",
                                  num_cores=NC, num_subcores=NS)


def _acc_init(s, zero_hbm, acc_sh):
    @pl.when(s < NS - 1)
    def _():
        pltpu.sync_copy(zero_hbm.at[pl.ds(s * ROWS_I, ROWS_I)],
                        acc_sh.at[pl.ds(s * ROWS_I, ROWS_I)])

    @pl.when(s == NS - 1)
    def _():
        pltpu.sync_copy(zero_hbm.at[pl.ds(15 * ROWS_I, ROWS_I_LAST)],
                        acc_sh.at[pl.ds(15 * ROWS_I, ROWS_I_LAST)])


def _acc_dump(c, s, acc_sh, out_hbm):
    @pl.when(s < NS - 1)
    def _():
        pltpu.sync_copy(acc_sh.at[pl.ds(s * ROWS_D, ROWS_D)],
                        out_hbm.at[c].at[pl.ds(s * ROWS_D, ROWS_D)])

    @pl.when(s == NS - 1)
    def _():
        pltpu.sync_copy(acc_sh.at[pl.ds(15 * ROWS_D, ROWS_D_LAST)],
                        out_hbm.at[c].at[pl.ds(15 * ROWS_D, ROWS_D_LAST)])


def _make_agg(F, NB):
    @functools.partial(
        pl.kernel,
        out_type=jax.ShapeDtypeStruct((NC, PAD_N, F), jnp.float32),
        mesh=_sc_mesh(),
        scratch_types=(
            [pltpu.VMEM((NCH, CH), jnp.int32),
             pltpu.VMEM((NCH, CH), jnp.int32)]
            + [pltpu.VMEM((CH, F), jnp.float32)] * NB
            + [pltpu.VMEM_SHARED((PAD_N, F), jnp.float32)]
            + [pltpu.SemaphoreType.DMA] * NB
        ),
        compiler_params=pltpu.CompilerParams(use_tc_tiling_on_sc=False),
    )
    def agg(y_hbm, src_hbm, dst_hbm, zero_hbm, out_hbm,
            sidx_v, didx_v, *rest):
        rows = rest[:NB]
        acc_sh = rest[NB]
        sems = rest[NB + 1:]
        c = lax.axis_index("c")
        s = lax.axis_index("s")
        w = c * NS + s
        pltpu.sync_copy(src_hbm.at[pl.ds(w * NCH, NCH)], sidx_v)
        pltpu.sync_copy(dst_hbm.at[pl.ds(w * NCH, NCH)], didx_v)
        _acc_init(s, zero_hbm, acc_sh)
        plsc.subcore_barrier()

        for b in range(NB - 1):
            pltpu.async_copy(y_hbm.at[sidx_v.at[b]], rows[b], sems[b])

        def body(g, carry):
            for b in range(NB):
                i = g * NB + b
                pltpu.make_async_copy(y_hbm.at[sidx_v.at[i]],
                                      rows[b], sems[b]).wait()

                @pl.when(i + NB - 1 < NCH)
                def _():
                    nb = (b + NB - 1) % NB
                    pltpu.async_copy(y_hbm.at[sidx_v.at[i + NB - 1]],
                                     rows[nb], sems[nb])

                pltpu.sync_copy(rows[b], acc_sh.at[didx_v.at[i]], add=True)
            return carry

        lax.fori_loop(0, NCH // NB, body, 0)
        plsc.subcore_barrier()
        _acc_dump(c, s, acc_sh, out_hbm)

    return agg


def _make_deg():
    @functools.partial(
        pl.kernel,
        out_type=jax.ShapeDtypeStruct((NC, PAD_N, 16), jnp.float32),
        mesh=_sc_mesh(),
        scratch_types=[
            pltpu.VMEM((NCH, CH), jnp.int32),
            pltpu.VMEM((CH, 16), jnp.float32),
            pltpu.VMEM_SHARED((PAD_N, 16), jnp.float32),
        ],
        compiler_params=pltpu.CompilerParams(use_tc_tiling_on_sc=False),
    )
    def deg(dst_hbm, ones_hbm, zero_hbm, out_hbm, didx_v, ones_v, acc_sh):
        c = lax.axis_index("c")
        s = lax.axis_index("s")
        w = c * NS + s
        pltpu.sync_copy(ones_hbm, ones_v)
        pltpu.sync_copy(dst_hbm.at[pl.ds(w * NCH, NCH)], didx_v)
        _acc_init(s, zero_hbm, acc_sh)
        plsc.subcore_barrier()

        def body(i, carry):
            pltpu.sync_copy(ones_v, acc_sh.at[didx_v.at[i]], add=True)
            return carry

        lax.fori_loop(0, NCH, body, 0)
        plsc.subcore_barrier()
        _acc_dump(c, s, acc_sh, out_hbm)

    return deg


def _tc_stage_a(latent, hs, Wl, Wt, degp):
    R = 400
    G = N // R

    def body(lat_ref, hs_ref, wl_ref, wt_ref, deg_ref, y_ref, dinv_ref):
        t = jnp.dot(lat_ref[...], wl_ref[...], preferred_element_type=jnp.float32)
        t = t + jnp.dot(hs_ref[...], wt_ref[...], preferred_element_type=jnp.float32)
        dp = deg_ref[...]
        deg = 1.0 + dp[0, :, 0:1] + dp[1, :, 0:1]
        dinv = lax.rsqrt(deg)
        y_ref[...] = t * dinv
        dinv_ref[...] = dinv

    return pl.pallas_call(
        body,
        grid=(G,),
        in_specs=[
            pl.BlockSpec((R, 256), lambda i: (i, 0)),
            pl.BlockSpec((R, 2), lambda i: (i, 0)),
            pl.BlockSpec((256, 128), lambda i: (0, 0)),
            pl.BlockSpec((2, 128), lambda i: (0, 0)),
            pl.BlockSpec((NC, R, 16), lambda i: (0, i, 0)),
        ],
        out_specs=[
            pl.BlockSpec((R, 128), lambda i: (i, 0)),
            pl.BlockSpec((R, 1), lambda i: (i, 0)),
        ],
        out_shape=[
            jax.ShapeDtypeStruct((N, 128), jnp.float32),
            jax.ShapeDtypeStruct((N, 1), jnp.float32),
        ],
    )(latent, hs, Wl, Wt, degp)


def _tc_stage_mid(acc, y, dinv, b, Wn, F, F2, F2_store):
    R = 400
    G = N // R

    def body(acc_ref, y_ref, dinv_ref, b_ref, w_ref, out_ref):
        a = acc_ref[...]
        dinv = dinv_ref[...]
        h = (a[0] + a[1] + y_ref[...]) * dinv + b_ref[...]
        h = jnp.maximum(h, 0.0)
        yn = jnp.dot(h, w_ref[...], preferred_element_type=jnp.float32) * dinv
        if F2_store != F2:
            yn = jnp.pad(yn, ((0, 0), (0, F2_store - F2)))
        out_ref[...] = yn

    return pl.pallas_call(
        body,
        grid=(G,),
        in_specs=[
            pl.BlockSpec((NC, R, F), lambda i: (0, i, 0)),
            pl.BlockSpec((R, F), lambda i: (i, 0)),
            pl.BlockSpec((R, 1), lambda i: (i, 0)),
            pl.BlockSpec((1, F), lambda i: (0, 0)),
            pl.BlockSpec((F, F2), lambda i: (0, 0)),
        ],
        out_specs=pl.BlockSpec((R, F2_store), lambda i: (i, 0)),
        out_shape=jax.ShapeDtypeStruct((N, F2_store), jnp.float32),
    )(acc, y, dinv, b, Wn)


def _tc_stage_final(acc, y, dinv, b3, Wc, bc):
    R = 400
    G = N // R

    def body(acc_ref, y_ref, dinv_ref, b_ref, w_ref, bc_ref, out_ref):
        a = acc_ref[...]
        dinv = dinv_ref[...]
        h = (a[0, :, 0:8] + a[1, :, 0:8] + y_ref[..., 0:8]) * dinv + b_ref[...]
        h = jnp.maximum(h, 0.0)
        out_ref[...] = (
            jnp.dot(h, w_ref[...], preferred_element_type=jnp.float32) + bc_ref[...]
        )

    return pl.pallas_call(
        body,
        grid=(G,),
        in_specs=[
            pl.BlockSpec((NC, R, 16), lambda i: (0, i, 0)),
            pl.BlockSpec((R, 16), lambda i: (i, 0)),
            pl.BlockSpec((R, 1), lambda i: (i, 0)),
            pl.BlockSpec((1, 8), lambda i: (0, 0)),
            pl.BlockSpec((8, 3), lambda i: (0, 0)),
            pl.BlockSpec((1, 3), lambda i: (0, 0)),
        ],
        out_specs=pl.BlockSpec((R, 3), lambda i: (i, 0)),
        out_shape=jax.ShapeDtypeStruct((N, 3), jnp.float32),
    )(acc, y, dinv, b3, Wc, bc)


def kernel(latent_features, history, is_subgoal, edge_data,
           W1, b1, W2, b2, W3, b3, Wc, bc):
    src = edge_data[0].astype(jnp.int32)
    dst = edge_data[1].astype(jnp.int32)
    n_pad_w = (E_PAD - E) // NW
    j = jnp.arange(n_pad_w, dtype=jnp.int32)
    pad_src = jnp.broadcast_to(N + (j % 16)[None, :], (NW, n_pad_w))
    pad_dst = (jnp.arange(NW, dtype=jnp.int32)[:, None] * n_pad_w + j[None, :]) % N
    src_p = jnp.concatenate(
        [src.reshape(NW, E // NW), pad_src], axis=1).reshape(E_PAD // CH, CH)
    dst_p = jnp.concatenate(
        [dst.reshape(NW, E // NW), pad_dst], axis=1).reshape(E_PAD // CH, CH)
    dst_deg = jnp.concatenate(
        [dst.reshape(NW, E // NW), pad_src], axis=1).reshape(E_PAD // CH, CH)

    zeros128 = jnp.zeros((N, 128), jnp.float32)
    zeros64 = jnp.zeros((N, 64), jnp.float32)
    zeros16 = jnp.zeros((N, 16), jnp.float32)
    ones16 = jnp.ones((CH, 16), jnp.float32)

    degp = _make_deg()(dst_deg, ones16, zeros16)

    hs = jnp.stack([history, is_subgoal], axis=1)
    y1, dinv = _tc_stage_a(latent_features, hs, W1[:256], W1[256:258], degp)

    pad16 = ((0, PAD_N - N), (0, 0))
    y1p = jnp.pad(y1, pad16)
    acc1 = _make_agg(128, 2)(y1p, src_p, dst_p, zeros128)
    y2 = _tc_stage_mid(acc1, y1, dinv, b1.reshape(1, -1), W2, 128, 64, 64)
    y2p = jnp.pad(y2, pad16)
    acc2 = _make_agg(64, 4)(y2p, src_p, dst_p, zeros64)
    y3 = _tc_stage_mid(acc2, y2, dinv, b2.reshape(1, -1), W3, 64, 8, 16)
    y3p = jnp.pad(y3, pad16)
    acc3 = _make_agg(16, 4)(y3p, src_p, dst_p, zeros16)
    out = _tc_stage_final(acc3, y3, dinv, b3.reshape(1, -1), Wc, bc.reshape(1, -1))
    return out


# speedup vs baseline: 1.8609x; 1.0092x over previous
import functools

import jax
import jax.numpy as jnp
from jax import lax
from jax.experimental import pallas as pl
from jax.experimental.pallas import tpu as pltpu
from jax.experimental.pallas import tpu_sc as plsc

N = 10000
E = 160000
PAD_N = 10016
NC, NS = 2, 16
NW = NC * NS
E_PAD = 163840
EPT = E_PAD // NW
CH = 128
NCH = EPT // CH
ROWS_I = 632
ROWS_I_LAST = N - 15 * ROWS_I
ROWS_D = 632
ROWS_D_LAST = PAD_N - 15 * ROWS_D


def _sc_mesh():
    return plsc.VectorSubcoreMesh(core_axis_name="c", subcore_axis_name="s",
                                  num_cores=NC, num_subcores=NS)


def _acc_init(s, zero_hbm, acc_sh):
    @pl.when(s < NS - 1)
    def _():
        pltpu.sync_copy(zero_hbm.at[pl.ds(s * ROWS_I, ROWS_I)],
                        acc_sh.at[pl.ds(s * ROWS_I, ROWS_I)])

    @pl.when(s == NS - 1)
    def _():
        pltpu.sync_copy(zero_hbm.at[pl.ds(15 * ROWS_I, ROWS_I_LAST)],
                        acc_sh.at[pl.ds(15 * ROWS_I, ROWS_I_LAST)])


def _acc_dump(c, s, acc_sh, out_hbm):
    @pl.when(s < NS - 1)
    def _():
        pltpu.sync_copy(acc_sh.at[pl.ds(s * ROWS_D, ROWS_D)],
                        out_hbm.at[c].at[pl.ds(s * ROWS_D, ROWS_D)])

    @pl.when(s == NS - 1)
    def _():
        pltpu.sync_copy(acc_sh.at[pl.ds(15 * ROWS_D, ROWS_D_LAST)],
                        out_hbm.at[c].at[pl.ds(15 * ROWS_D, ROWS_D_LAST)])


def _make_agg(F, NB):
    @functools.partial(
        pl.kernel,
        out_type=jax.ShapeDtypeStruct((NC, PAD_N, F), jnp.float32),
        mesh=_sc_mesh(),
        scratch_types=(
            [pltpu.VMEM((NCH, CH), jnp.int32),
             pltpu.VMEM((NCH, CH), jnp.int32)]
            + [pltpu.VMEM((CH, F), jnp.float32)] * NB
            + [pltpu.VMEM_SHARED((PAD_N, F), jnp.float32)]
            + [pltpu.SemaphoreType.DMA] * NB
        ),
        compiler_params=pltpu.CompilerParams(use_tc_tiling_on_sc=False),
    )
    def agg(y_hbm, src_hbm, dst_hbm, zero_hbm, out_hbm,
            sidx_v, didx_v, *rest):
        rows = rest[:NB]
        acc_sh = rest[NB]
        sems = rest[NB + 1:]
        c = lax.axis_index("c")
        s = lax.axis_index("s")
        w = c * NS + s
        pltpu.sync_copy(src_hbm.at[pl.ds(w * NCH, NCH)], sidx_v)
        pltpu.sync_copy(dst_hbm.at[pl.ds(w * NCH, NCH)], didx_v)
        _acc_init(s, zero_hbm, acc_sh)
        plsc.subcore_barrier()

        for b in range(NB - 1):
            pltpu.async_copy(y_hbm.at[sidx_v.at[b]], rows[b], sems[b])

        def body(g, carry):
            for b in range(NB):
                i = g * NB + b
                pltpu.make_async_copy(y_hbm.at[sidx_v.at[i]],
                                      rows[b], sems[b]).wait()

                @pl.when(i + NB - 1 < NCH)
                def _():
                    nb = (b + NB - 1) % NB
                    pltpu.async_copy(y_hbm.at[sidx_v.at[i + NB - 1]],
                                     rows[nb], sems[nb])

                pltpu.sync_copy(rows[b], acc_sh.at[didx_v.at[i]], add=True)
            return carry

        lax.fori_loop(0, NCH // NB, body, 0)
        plsc.subcore_barrier()
        _acc_dump(c, s, acc_sh, out_hbm)

    return agg


def _make_deg():
    @functools.partial(
        pl.kernel,
        out_type=jax.ShapeDtypeStruct((NC, PAD_N, 16), jnp.float32),
        mesh=_sc_mesh(),
        scratch_types=[
            pltpu.VMEM((NCH, CH), jnp.int32),
            pltpu.VMEM((CH, 16), jnp.float32),
            pltpu.VMEM_SHARED((PAD_N, 16), jnp.float32),
        ],
        compiler_params=pltpu.CompilerParams(use_tc_tiling_on_sc=False),
    )
    def deg(dst_hbm, ones_hbm, zero_hbm, out_hbm, didx_v, ones_v, acc_sh):
        c = lax.axis_index("c")
        s = lax.axis_index("s")
        w = c * NS + s
        pltpu.sync_copy(ones_hbm, ones_v)
        pltpu.sync_copy(dst_hbm.at[pl.ds(w * NCH, NCH)], didx_v)
        _acc_init(s, zero_hbm, acc_sh)
        plsc.subcore_barrier()

        def body(i, carry):
            pltpu.sync_copy(ones_v, acc_sh.at[didx_v.at[i]], add=True)
            return carry

        lax.fori_loop(0, NCH, body, 0)
        plsc.subcore_barrier()
        _acc_dump(c, s, acc_sh, out_hbm)

    return deg


def _tc_stage_a(latent, hs, Wl, Wt, degp):
    R = 400
    G = N // R

    def body(lat_ref, hs_ref, wl_ref, wt_ref, deg_ref, y_ref, dinv_ref):
        t = jnp.dot(lat_ref[...], wl_ref[...], preferred_element_type=jnp.float32)
        t = t + jnp.dot(hs_ref[...], wt_ref[...], preferred_element_type=jnp.float32)
        dp = deg_ref[...]
        deg = 1.0 + dp[0, :, 0:1] + dp[1, :, 0:1]
        dinv = lax.rsqrt(deg)
        y_ref[...] = t * dinv
        dinv_ref[...] = dinv

    return pl.pallas_call(
        body,
        grid=(G,),
        in_specs=[
            pl.BlockSpec((R, 256), lambda i: (i, 0)),
            pl.BlockSpec((R, 2), lambda i: (i, 0)),
            pl.BlockSpec((256, 128), lambda i: (0, 0)),
            pl.BlockSpec((2, 128), lambda i: (0, 0)),
            pl.BlockSpec((NC, R, 16), lambda i: (0, i, 0)),
        ],
        out_specs=[
            pl.BlockSpec((R, 128), lambda i: (i, 0)),
            pl.BlockSpec((R, 1), lambda i: (i, 0)),
        ],
        out_shape=[
            jax.ShapeDtypeStruct((N, 128), jnp.float32),
            jax.ShapeDtypeStruct((N, 1), jnp.float32),
        ],
    )(latent, hs, Wl, Wt, degp)


def _tc_stage_mid(acc, y, dinv, b, Wn, F, F2, F2_store):
    R = 400
    G = N // R

    def body(acc_ref, y_ref, dinv_ref, b_ref, w_ref, out_ref):
        a = acc_ref[...]
        dinv = dinv_ref[...]
        h = (a[0] + a[1] + y_ref[...]) * dinv + b_ref[...]
        h = jnp.maximum(h, 0.0)
        yn = jnp.dot(h, w_ref[...], preferred_element_type=jnp.float32) * dinv
        if F2_store != F2:
            yn = jnp.pad(yn, ((0, 0), (0, F2_store - F2)))
        out_ref[...] = yn

    return pl.pallas_call(
        body,
        grid=(G,),
        in_specs=[
            pl.BlockSpec((NC, R, F), lambda i: (0, i, 0)),
            pl.BlockSpec((R, F), lambda i: (i, 0)),
            pl.BlockSpec((R, 1), lambda i: (i, 0)),
            pl.BlockSpec((1, F), lambda i: (0, 0)),
            pl.BlockSpec((F, F2), lambda i: (0, 0)),
        ],
        out_specs=pl.BlockSpec((R, F2_store), lambda i: (i, 0)),
        out_shape=jax.ShapeDtypeStruct((N, F2_store), jnp.float32),
    )(acc, y, dinv, b, Wn)


def _tc_stage_final(acc, y, dinv, b3, Wc, bc):
    R = 400
    G = N // R

    def body(acc_ref, y_ref, dinv_ref, b_ref, w_ref, bc_ref, out_ref):
        a = acc_ref[...]
        dinv = dinv_ref[...]
        h = (a[0, :, 0:8] + a[1, :, 0:8] + y_ref[..., 0:8]) * dinv + b_ref[...]
        h = jnp.maximum(h, 0.0)
        out_ref[...] = (
            jnp.dot(h, w_ref[...], preferred_element_type=jnp.float32) + bc_ref[...]
        )

    return pl.pallas_call(
        body,
        grid=(G,),
        in_specs=[
            pl.BlockSpec((NC, R, 16), lambda i: (0, i, 0)),
            pl.BlockSpec((R, 16), lambda i: (i, 0)),
            pl.BlockSpec((R, 1), lambda i: (i, 0)),
            pl.BlockSpec((1, 8), lambda i: (0, 0)),
            pl.BlockSpec((8, 3), lambda i: (0, 0)),
            pl.BlockSpec((1, 3), lambda i: (0, 0)),
        ],
        out_specs=pl.BlockSpec((R, 3), lambda i: (i, 0)),
        out_shape=jax.ShapeDtypeStruct((N, 3), jnp.float32),
    )(acc, y, dinv, b3, Wc, bc)


def kernel(latent_features, history, is_subgoal, edge_data,
           W1, b1, W2, b2, W3, b3, Wc, bc):
    src = edge_data[0].astype(jnp.int32)
    dst = edge_data[1].astype(jnp.int32)
    j = jnp.arange(E_PAD - E, dtype=jnp.int32)
    pad_src = N + (j % 16)
    src_p = jnp.concatenate([src, pad_src]).reshape(E_PAD // CH, CH)
    dst_p = jnp.concatenate([dst, j % N]).reshape(E_PAD // CH, CH)
    dst_deg = jnp.concatenate([dst, pad_src]).reshape(E_PAD // CH, CH)
    src_p, dst_p, dst_deg = lax.optimization_barrier((src_p, dst_p, dst_deg))

    zeros128, zeros64, zeros16, ones16 = lax.optimization_barrier((
        jnp.zeros((N, 128), jnp.float32),
        jnp.zeros((N, 64), jnp.float32),
        jnp.zeros((N, 16), jnp.float32),
        jnp.ones((CH, 16), jnp.float32),
    ))

    degp = _make_deg()(dst_deg, ones16, zeros16)

    hs = jnp.stack([history, is_subgoal], axis=1)
    y1, dinv = _tc_stage_a(latent_features, hs, W1[:256], W1[256:258], degp)

    pad16 = ((0, PAD_N - N), (0, 0))
    y1p = jnp.pad(y1, pad16)
    acc1 = _make_agg(128, 2)(y1p, src_p, dst_p, zeros128)
    y2 = _tc_stage_mid(acc1, y1, dinv, b1.reshape(1, -1), W2, 128, 64, 64)
    y2p = jnp.pad(y2, pad16)
    acc2 = _make_agg(64, 4)(y2p, src_p, dst_p, zeros64)
    y3 = _tc_stage_mid(acc2, y2, dinv, b2.reshape(1, -1), W3, 64, 8, 16)
    y3p = jnp.pad(y3, pad16)
    acc3 = _make_agg(16, 4)(y3p, src_p, dst_p, zeros16)
    out = _tc_stage_final(acc3, y3, dinv, b3.reshape(1, -1), Wc, bc.reshape(1, -1))
    return out


# speedup vs baseline: 2.1695x; 1.1658x over previous
import functools

import jax
import jax.numpy as jnp
from jax import lax
from jax.experimental import pallas as pl
from jax.experimental.pallas import tpu as pltpu
from jax.experimental.pallas import tpu_sc as plsc

N = 10000
E = 160000
PAD_N = 10016
NC, NS = 2, 16
NW = NC * NS
E_PAD = 163840
EPT = E_PAD // NW
CH = 128
NCH = EPT // CH
ROWS_I = 632
ROWS_I_LAST = N - 15 * ROWS_I
ROWS_D = 632
ROWS_D_LAST = PAD_N - 15 * ROWS_D


def _sc_mesh():
    return plsc.VectorSubcoreMesh(core_axis_name="c", subcore_axis_name="s",
                                  num_cores=NC, num_subcores=NS)


def _acc_init(s, zero_hbm, acc_sh):
    @pl.when(s < NS - 1)
    def _():
        pltpu.sync_copy(zero_hbm.at[pl.ds(s * ROWS_I, ROWS_I)],
                        acc_sh.at[pl.ds(s * ROWS_I, ROWS_I)])

    @pl.when(s == NS - 1)
    def _():
        pltpu.sync_copy(zero_hbm.at[pl.ds(15 * ROWS_I, ROWS_I_LAST)],
                        acc_sh.at[pl.ds(15 * ROWS_I, ROWS_I_LAST)])


def _acc_dump(c, s, acc_sh, out_hbm):
    @pl.when(s < NS - 1)
    def _():
        pltpu.sync_copy(acc_sh.at[pl.ds(s * ROWS_D, ROWS_D)],
                        out_hbm.at[c].at[pl.ds(s * ROWS_D, ROWS_D)])

    @pl.when(s == NS - 1)
    def _():
        pltpu.sync_copy(acc_sh.at[pl.ds(15 * ROWS_D, ROWS_D_LAST)],
                        out_hbm.at[c].at[pl.ds(15 * ROWS_D, ROWS_D_LAST)])


def _make_agg(F, NB):
    @functools.partial(
        pl.kernel,
        out_type=jax.ShapeDtypeStruct((NC, PAD_N, F), jnp.float32),
        mesh=_sc_mesh(),
        scratch_types=(
            [pltpu.VMEM((NCH, CH), jnp.int32),
             pltpu.VMEM((NCH, CH), jnp.int32)]
            + [pltpu.VMEM((CH, F), jnp.float32)] * NB
            + [pltpu.VMEM_SHARED((PAD_N, F), jnp.float32)]
            + [pltpu.SemaphoreType.DMA] * NB
        ),
        compiler_params=pltpu.CompilerParams(use_tc_tiling_on_sc=False),
    )
    def agg(y_hbm, src_hbm, dst_hbm, zero_hbm, out_hbm,
            sidx_v, didx_v, *rest):
        rows = rest[:NB]
        acc_sh = rest[NB]
        sems = rest[NB + 1:]
        c = lax.axis_index("c")
        s = lax.axis_index("s")
        w = c * NS + s
        pltpu.sync_copy(src_hbm.at[pl.ds(w * NCH, NCH)], sidx_v)
        pltpu.sync_copy(dst_hbm.at[pl.ds(w * NCH, NCH)], didx_v)
        _acc_init(s, zero_hbm, acc_sh)
        plsc.subcore_barrier()

        for b in range(NB - 1):
            pltpu.async_copy(y_hbm.at[sidx_v.at[b]], rows[b], sems[b])

        def body(g, carry):
            for b in range(NB):
                i = g * NB + b
                pltpu.make_async_copy(y_hbm.at[sidx_v.at[i]],
                                      rows[b], sems[b]).wait()

                @pl.when(i + NB - 1 < NCH)
                def _():
                    nb = (b + NB - 1) % NB
                    pltpu.async_copy(y_hbm.at[sidx_v.at[i + NB - 1]],
                                     rows[nb], sems[nb])

                pltpu.sync_copy(rows[b], acc_sh.at[didx_v.at[i]], add=True)
            return carry

        lax.fori_loop(0, NCH // NB, body, 0)
        plsc.subcore_barrier()
        _acc_dump(c, s, acc_sh, out_hbm)

    return agg


def _make_deg():
    @functools.partial(
        pl.kernel,
        out_type=jax.ShapeDtypeStruct((NC, PAD_N, 16), jnp.float32),
        mesh=_sc_mesh(),
        scratch_types=[
            pltpu.VMEM((NCH, CH), jnp.int32),
            pltpu.VMEM((CH, 16), jnp.float32),
            pltpu.VMEM_SHARED((PAD_N, 16), jnp.float32),
        ],
        compiler_params=pltpu.CompilerParams(use_tc_tiling_on_sc=False),
    )
    def deg(dst_hbm, ones_hbm, zero_hbm, out_hbm, didx_v, ones_v, acc_sh):
        c = lax.axis_index("c")
        s = lax.axis_index("s")
        w = c * NS + s
        pltpu.sync_copy(ones_hbm, ones_v)
        pltpu.sync_copy(dst_hbm.at[pl.ds(w * NCH, NCH)], didx_v)
        _acc_init(s, zero_hbm, acc_sh)
        plsc.subcore_barrier()

        def body(i, carry):
            pltpu.sync_copy(ones_v, acc_sh.at[didx_v.at[i]], add=True)
            return carry

        lax.fori_loop(0, NCH, body, 0)
        plsc.subcore_barrier()
        _acc_dump(c, s, acc_sh, out_hbm)

    return deg


def _tc_stage_a1(latent, hs, Wl, Wt):
    R = 2000
    G = N // R

    def body(lat_ref, hs_ref, wl_ref, wt_ref, t_ref):
        t = jnp.dot(lat_ref[...], wl_ref[...], preferred_element_type=jnp.float32)
        t_ref[...] = t + jnp.dot(hs_ref[...], wt_ref[...],
                                 preferred_element_type=jnp.float32)

    return pl.pallas_call(
        body,
        grid=(G,),
        in_specs=[
            pl.BlockSpec((R, 256), lambda i: (i, 0)),
            pl.BlockSpec((R, 2), lambda i: (i, 0)),
            pl.BlockSpec((256, 128), lambda i: (0, 0)),
            pl.BlockSpec((2, 128), lambda i: (0, 0)),
        ],
        out_specs=pl.BlockSpec((R, 128), lambda i: (i, 0)),
        out_shape=jax.ShapeDtypeStruct((N, 128), jnp.float32),
    )(latent, hs, Wl, Wt)


def _tc_stage_a2(t1, degp):
    R = 2000
    G = N // R

    def body(t_ref, deg_ref, y_ref, dinv_ref):
        dp = deg_ref[...]
        deg = 1.0 + dp[0, :, 0:1] + dp[1, :, 0:1]
        dinv = lax.rsqrt(deg)
        y_ref[...] = t_ref[...] * dinv
        dinv_ref[...] = dinv

    return pl.pallas_call(
        body,
        grid=(G,),
        in_specs=[
            pl.BlockSpec((R, 128), lambda i: (i, 0)),
            pl.BlockSpec((NC, R, 16), lambda i: (0, i, 0)),
        ],
        out_specs=[
            pl.BlockSpec((R, 128), lambda i: (i, 0)),
            pl.BlockSpec((R, 1), lambda i: (i, 0)),
        ],
        out_shape=[
            jax.ShapeDtypeStruct((N, 128), jnp.float32),
            jax.ShapeDtypeStruct((N, 1), jnp.float32),
        ],
    )(t1, degp)


def _tc_stage_mid(acc, y, dinv, b, Wn, F, F2, F2_store):
    R = 2000
    G = N // R

    def body(acc_ref, y_ref, dinv_ref, b_ref, w_ref, out_ref):
        a = acc_ref[...]
        dinv = dinv_ref[...]
        h = (a[0] + a[1] + y_ref[...]) * dinv + b_ref[...]
        h = jnp.maximum(h, 0.0)
        yn = jnp.dot(h, w_ref[...], preferred_element_type=jnp.float32) * dinv
        if F2_store != F2:
            yn = jnp.pad(yn, ((0, 0), (0, F2_store - F2)))
        out_ref[...] = yn

    return pl.pallas_call(
        body,
        grid=(G,),
        in_specs=[
            pl.BlockSpec((NC, R, F), lambda i: (0, i, 0)),
            pl.BlockSpec((R, F), lambda i: (i, 0)),
            pl.BlockSpec((R, 1), lambda i: (i, 0)),
            pl.BlockSpec((1, F), lambda i: (0, 0)),
            pl.BlockSpec((F, F2), lambda i: (0, 0)),
        ],
        out_specs=pl.BlockSpec((R, F2_store), lambda i: (i, 0)),
        out_shape=jax.ShapeDtypeStruct((N, F2_store), jnp.float32),
    )(acc, y, dinv, b, Wn)


def _tc_stage_final(acc, y, dinv, b3, Wc, bc):
    R = 2000
    G = N // R

    def body(acc_ref, y_ref, dinv_ref, b_ref, w_ref, bc_ref, out_ref):
        a = acc_ref[...]
        dinv = dinv_ref[...]
        h = (a[0, :, 0:8] + a[1, :, 0:8] + y_ref[..., 0:8]) * dinv + b_ref[...]
        h = jnp.maximum(h, 0.0)
        out_ref[...] = (
            jnp.dot(h, w_ref[...], preferred_element_type=jnp.float32) + bc_ref[...]
        )

    return pl.pallas_call(
        body,
        grid=(G,),
        in_specs=[
            pl.BlockSpec((NC, R, 16), lambda i: (0, i, 0)),
            pl.BlockSpec((R, 16), lambda i: (i, 0)),
            pl.BlockSpec((R, 1), lambda i: (i, 0)),
            pl.BlockSpec((1, 8), lambda i: (0, 0)),
            pl.BlockSpec((8, 3), lambda i: (0, 0)),
            pl.BlockSpec((1, 3), lambda i: (0, 0)),
        ],
        out_specs=pl.BlockSpec((R, 3), lambda i: (i, 0)),
        out_shape=jax.ShapeDtypeStruct((N, 3), jnp.float32),
    )(acc, y, dinv, b3, Wc, bc)


def kernel(latent_features, history, is_subgoal, edge_data,
           W1, b1, W2, b2, W3, b3, Wc, bc):
    src = edge_data[0].astype(jnp.int32)
    dst = edge_data[1].astype(jnp.int32)
    j = jnp.arange(E_PAD - E, dtype=jnp.int32)
    pad_src = N + (j % 16)
    src_p = jnp.concatenate([src, pad_src]).reshape(E_PAD // CH, CH)
    dst_p = jnp.concatenate([dst, j % N]).reshape(E_PAD // CH, CH)
    dst_deg = jnp.concatenate([dst, pad_src]).reshape(E_PAD // CH, CH)
    src_p, dst_p, dst_deg = lax.optimization_barrier((src_p, dst_p, dst_deg))

    zeros128, zeros64, zeros16, ones16 = lax.optimization_barrier((
        jnp.zeros((N, 128), jnp.float32),
        jnp.zeros((N, 64), jnp.float32),
        jnp.zeros((N, 16), jnp.float32),
        jnp.ones((CH, 16), jnp.float32),
    ))

    degp = _make_deg()(dst_deg, ones16, zeros16)

    hs = jnp.stack([history, is_subgoal], axis=1)
    t1 = _tc_stage_a1(latent_features, hs, W1[:256], W1[256:258])
    y1, dinv = _tc_stage_a2(t1, degp)

    pad16 = ((0, PAD_N - N), (0, 0))
    y1p = jnp.pad(y1, pad16)
    acc1 = _make_agg(128, 2)(y1p, src_p, dst_p, zeros128)
    y2 = _tc_stage_mid(acc1, y1, dinv, b1.reshape(1, -1), W2, 128, 64, 64)
    y2p = jnp.pad(y2, pad16)
    acc2 = _make_agg(64, 4)(y2p, src_p, dst_p, zeros64)
    y3 = _tc_stage_mid(acc2, y2, dinv, b2.reshape(1, -1), W3, 64, 8, 16)
    y3p = jnp.pad(y3, pad16)
    acc3 = _make_agg(16, 4)(y3p, src_p, dst_p, zeros16)
    out = _tc_stage_final(acc3, y3, dinv, b3.reshape(1, -1), Wc, bc.reshape(1, -1))
    return out
